# Initial kernel scaffold; baseline (speedup 1.0000x reference)
import jax
import jax.numpy as jnp
from jax.experimental import pallas as pl


def kernel(block_outputs, pos_edge_index, neg_edge_index):
    raise NotImplementedError("write your pallas kernel here")


# speedup vs baseline: 2.9024x; 2.9024x over previous
import functools

import jax
import jax.numpy as jnp
from jax import lax
from jax.experimental import pallas as pl
from jax.experimental.pallas import tpu as pltpu
from jax.experimental.pallas import tpu_sc as plsc

_LANES = 16


def _scores_sc_kernel(n_workers: int, e_pos: int, e_neg: int, d: int,
                      chunk: int):
    per_w_pos = e_pos // n_workers
    per_w_neg = e_neg // n_workers
    groups = chunk // _LANES

    def body(table_hbm, pos_hbm, neg_hbm, out_hbm,
             src_idx_v, dst_idx_v, src_rows_v, dst_rows_v,
             scores_v, sem):
        cid = lax.axis_index("c")
        sid = lax.axis_index("s")
        n_cores = lax.axis_size("c")
        wid = sid * n_cores + cid

        def do_range(eidx_hbm, e_half, per_w, out_base):
            base = wid * per_w
            n_chunks = per_w // chunk

            def chunk_body(i, _):
                off = pl.multiple_of(base + i * chunk, 8)
                off2 = pl.multiple_of(e_half + base + i * chunk, 8)
                pltpu.sync_copy(eidx_hbm.at[pl.ds(off, chunk)], src_idx_v)
                pltpu.sync_copy(eidx_hbm.at[pl.ds(off2, chunk)], dst_idx_v)
                cp1 = pltpu.async_copy(table_hbm.at[src_idx_v], src_rows_v,
                                       sem)
                cp2 = pltpu.async_copy(table_hbm.at[dst_idx_v], dst_rows_v,
                                       sem)
                cp1.wait()
                cp2.wait()

                def group_body(g, _):
                    e0 = g * _LANES
                    lane = lax.iota(jnp.int32, _LANES)
                    tot = jnp.zeros((_LANES,), jnp.float32)
                    for e16 in range(_LANES):
                        e = e0 + e16
                        acc = jnp.zeros((_LANES,), jnp.float32)
                        for j in range(d // _LANES):
                            s = src_rows_v[e, pl.ds(j * _LANES, _LANES)]
                            t = dst_rows_v[e, pl.ds(j * _LANES, _LANES)]
                            acc = acc + s * t
                        sc = jnp.sum(acc)
                        tot = jnp.where(lane == e16, sc, tot)
                    scores_v[pl.ds(e0, _LANES)] = tot
                    return 0

                lax.fori_loop(0, groups, group_body, 0)
                out_off = pl.multiple_of(out_base + base + i * chunk, 8)
                pltpu.sync_copy(scores_v, out_hbm.at[pl.ds(out_off, chunk)])
                return 0

            lax.fori_loop(0, n_chunks, chunk_body, 0)

        do_range(pos_hbm, e_pos, per_w_pos, 0)
        do_range(neg_hbm, e_neg, per_w_neg, e_pos)

    return body


def _bce_tc_kernel(pos_rows: int, total: int):
    def body(s_ref, o_ref):
        s = s_ref[...]
        rows = lax.broadcasted_iota(jnp.int32, s.shape, 0)
        label = jnp.where(rows < pos_rows, 1.0, 0.0)
        t = (jnp.maximum(s, 0.0) - s * label
             + jnp.log1p(jnp.exp(-jnp.abs(s))))
        o_ref[...] = (jnp.sum(t) / total).reshape(1, 1)
    return body


@jax.jit
def kernel(block_outputs, pos_edge_index, neg_edge_index):
    n, d = block_outputs.shape
    e_pos = pos_edge_index.shape[1]
    e_neg = neg_edge_index.shape[1]
    total = e_pos + e_neg

    info = plsc.get_sparse_core_info()
    nw = info.num_cores * info.num_subcores
    chunk = 80
    assert e_pos % (nw * chunk) == 0 and e_neg % (nw * chunk) == 0
    assert d % _LANES == 0

    mesh = plsc.VectorSubcoreMesh(core_axis_name="c", subcore_axis_name="s")
    scores = pl.kernel(
        _scores_sc_kernel(nw, e_pos, e_neg, d, chunk),
        out_type=jax.ShapeDtypeStruct((total,), jnp.float32),
        mesh=mesh,
        compiler_params=pltpu.CompilerParams(needs_layout_passes=False),
        scratch_types=[
            pltpu.VMEM((chunk,), jnp.int32),
            pltpu.VMEM((chunk,), jnp.int32),
            pltpu.VMEM((chunk, d), jnp.float32),
            pltpu.VMEM((chunk, d), jnp.float32),
            pltpu.VMEM((chunk,), jnp.float32),
            pltpu.SemaphoreType.DMA,
        ],
    )(block_outputs, pos_edge_index.reshape(-1), neg_edge_index.reshape(-1))

    cols = 128
    rows = total // cols
    pos_rows = e_pos // cols
    loss = pl.pallas_call(
        _bce_tc_kernel(pos_rows, total),
        out_shape=jax.ShapeDtypeStruct((1, 1), jnp.float32),
    )(scores.reshape(rows, cols))
    return loss[0, 0]
